# Initial kernel scaffold; baseline (speedup 1.0000x reference)
import jax
import jax.numpy as jnp
from jax.experimental import pallas as pl


def kernel(points, idx):
    raise NotImplementedError("write your pallas kernel here")


# speedup vs baseline: 707.3282x; 707.3282x over previous
import functools

import jax
import jax.numpy as jnp
from jax import lax
from jax.experimental import pallas as pl
from jax.experimental.pallas import tpu as pltpu
from jax.experimental.pallas import tpu_sc as plsc


def _grouping_body(points_hbm, idx_hbm, out_hbm, idx_v, row_v, out_v):
    B, C, N = points_hbm.shape
    _, QS = idx_hbm.shape
    info = plsc.get_sparse_core_info()
    NC, NS, L = info.num_cores, info.num_subcores, info.num_lanes
    NW = NC * NS
    tiles_per_b = NW // B
    c_per_tile = C // tiles_per_b

    wid = lax.axis_index("s") * NC + lax.axis_index("c")
    b = wid // tiles_per_b
    c0 = (wid % tiles_per_b) * c_per_tile

    pltpu.sync_copy(idx_hbm.at[b], idx_v)

    def gather_step(i, _):
        iv = idx_v[pl.ds(i * L, L)]
        out_v[pl.ds(i * L, L)] = plsc.load_gather(row_v, [iv])
        return 0

    for j in range(c_per_tile):
        cc = c0 + j
        pltpu.sync_copy(points_hbm.at[b, cc], row_v)
        lax.fori_loop(0, QS // L, gather_step, 0)
        pltpu.sync_copy(out_v, out_hbm.at[b, cc])


def _make_grouping(B, C, N, QS):
    return functools.partial(
        pl.kernel,
        out_type=jax.ShapeDtypeStruct((B, C, QS), jnp.float32),
        mesh=plsc.VectorSubcoreMesh(core_axis_name="c", subcore_axis_name="s"),
        compiler_params=pltpu.CompilerParams(needs_layout_passes=False),
        scratch_types=[
            pltpu.VMEM((QS,), jnp.int32),
            pltpu.VMEM((N,), jnp.float32),
            pltpu.VMEM((QS,), jnp.float32),
        ],
    )(_grouping_body)


@jax.jit
def kernel(points, idx):
    B, C, N = points.shape
    _, npoint, nsample = idx.shape
    QS = npoint * nsample
    idx_flat = idx.astype(jnp.int32).reshape(B, QS)
    out = _make_grouping(B, C, N, QS)(points, idx_flat)
    return out.reshape(B, C, npoint, nsample)


# speedup vs baseline: 834.3802x; 1.1796x over previous
import functools

import jax
import jax.numpy as jnp
from jax import lax
from jax.experimental import pallas as pl
from jax.experimental.pallas import tpu as pltpu
from jax.experimental.pallas import tpu_sc as plsc


def _grouping_body(points_hbm, idx_hbm, out_hbm, idx_v, row_v, out_v):
    B, C, N = points_hbm.shape
    _, QS = idx_hbm.shape
    info = plsc.get_sparse_core_info()
    NC, NS, L = info.num_cores, info.num_subcores, info.num_lanes
    NW = NC * NS
    tiles_per_b = NW // B
    c_per_tile = C // tiles_per_b

    wid = lax.axis_index("s") * NC + lax.axis_index("c")
    b = wid // tiles_per_b
    c0 = (wid % tiles_per_b) * c_per_tile

    pltpu.sync_copy(idx_hbm.at[b], idx_v)

    U = 8

    def gather_step(i, _):
        base = i * (L * U)
        for u in range(U):
            off = base + u * L
            iv = idx_v[pl.ds(off, L)]
            out_v[pl.ds(off, L)] = plsc.load_gather(row_v, [iv])
        return 0

    for j in range(c_per_tile):
        cc = c0 + j
        pltpu.sync_copy(points_hbm.at[b, cc], row_v)
        lax.fori_loop(0, QS // (L * U), gather_step, 0)
        pltpu.sync_copy(out_v, out_hbm.at[b, cc])


def _make_grouping(B, C, N, QS):
    return functools.partial(
        pl.kernel,
        out_type=jax.ShapeDtypeStruct((B, C, QS), jnp.float32),
        mesh=plsc.VectorSubcoreMesh(core_axis_name="c", subcore_axis_name="s"),
        compiler_params=pltpu.CompilerParams(needs_layout_passes=False),
        scratch_types=[
            pltpu.VMEM((QS,), jnp.int32),
            pltpu.VMEM((N,), jnp.float32),
            pltpu.VMEM((QS,), jnp.float32),
        ],
    )(_grouping_body)


@jax.jit
def kernel(points, idx):
    B, C, N = points.shape
    _, npoint, nsample = idx.shape
    QS = npoint * nsample
    idx_flat = idx.astype(jnp.int32).reshape(B, QS)
    out = _make_grouping(B, C, N, QS)(points, idx_flat)
    return out.reshape(B, C, npoint, nsample)
